# baseline (speedup 1.0000x reference)
import jax
import jax.numpy as jnp
from jax import lax
from jax.experimental import pallas as pl
from jax.experimental.pallas import tpu as pltpu
from jax.experimental.pallas import tpu_sc as plsc

_N = 10000
_E = 320000
_D = 128
_NC = 2
_NS = 16
_NW = _NC * _NS
_EPW = _E // _NW
_CH = 128
_NCHUNK = _EPW // _CH
_TAIL = _EPW - _NCHUNK * _CH
_RPT = 640
_NP = _RPT * _NS

_ROWS = 1000
_GRID = _N // _ROWS


def _make_edge_accum(with_deg):
  mesh = plsc.VectorSubcoreMesh(core_axis_name="c", subcore_axis_name="s")
  out_type = [jax.ShapeDtypeStruct((_NC, _NP, _D), jnp.float32)]
  scratch = [
      pltpu.VMEM((_CH,), jnp.int32),
      pltpu.VMEM((_CH,), jnp.int32),
      pltpu.VMEM((_CH,), jnp.int32),
      pltpu.VMEM((_CH,), jnp.int32),
      pltpu.VMEM((_CH, _D), jnp.float32),
      pltpu.VMEM((_CH, _D), jnp.float32),
      pltpu.VMEM_SHARED((_NP, _D), jnp.float32),
      pltpu.SemaphoreType.DMA,
      pltpu.SemaphoreType.DMA,
      pltpu.SemaphoreType.DMA,
      pltpu.SemaphoreType.DMA,
      pltpu.VMEM((_TAIL,), jnp.int32),
      pltpu.VMEM((_TAIL,), jnp.int32),
      pltpu.VMEM((_TAIL, _D), jnp.float32),
  ]
  if with_deg:
    out_type.append(jax.ShapeDtypeStruct((_NC, _NP, _D), jnp.float32))

  def body(table, src_r, dst_r, zeros_d, ones_d, *rest):
    if with_deg:
      (out_sum, out_deg, src_a, dst_a, src_b, dst_b,
       rows_a, rows_b, acc, sem_a, sem_b, ssem_a, ssem_b,
       src_t, dst_t, rows_t) = rest
    else:
      (out_sum, src_a, dst_a, src_b, dst_b,
       rows_a, rows_b, acc, sem_a, sem_b, ssem_a, ssem_b,
       src_t, dst_t, rows_t) = rest
    c = lax.axis_index("c")
    s = lax.axis_index("s")
    wid = c * _NS + s
    r0 = s * _RPT
    e0 = wid * _EPW

    def zero_acc():
      pltpu.sync_copy(zeros_d, rows_a)
      for k in range(_RPT // _CH):
        pltpu.sync_copy(rows_a, acc.at[pl.ds(r0 + k * _CH, _CH)])

    def writeout(dst_hbm):
      for k in range(_RPT // _CH):
        pltpu.sync_copy(acc.at[pl.ds(r0 + k * _CH, _CH)], rows_a)
        pltpu.sync_copy(rows_a, dst_hbm.at[c, pl.ds(r0 + k * _CH, _CH)])

    zero_acc()
    plsc.subcore_barrier()

    def step(j, s_cur, d_cur, r_cur, g_cur, sc_cur,
             s_nxt, d_nxt, r_nxt, g_nxt, sc_nxt):
      pltpu.make_async_copy(table.at[s_cur], r_cur, g_cur).wait()
      pltpu.async_copy(r_cur, acc.at[d_cur], sc_cur, add=True)

      @pl.when(j < _NCHUNK - 1)
      def _():
        @pl.when(j >= 1)
        def _():
          pltpu.make_async_copy(r_nxt, acc.at[d_nxt], sc_nxt).wait()

        nbase = e0 + (j + 1) * _CH
        pltpu.sync_copy(src_r.at[pl.ds(nbase, _CH)], s_nxt)
        pltpu.sync_copy(dst_r.at[pl.ds(nbase, _CH)], d_nxt)
        pltpu.async_copy(table.at[s_nxt], r_nxt, g_nxt)

    def chunk(j, carry):
      @pl.when(j % 2 == 0)
      def _():
        step(j, src_a, dst_a, rows_a, sem_a, ssem_a,
             src_b, dst_b, rows_b, sem_b, ssem_b)

      @pl.when(j % 2 == 1)
      def _():
        step(j, src_b, dst_b, rows_b, sem_b, ssem_b,
             src_a, dst_a, rows_a, sem_a, ssem_a)

      return carry

    pltpu.sync_copy(src_r.at[pl.ds(e0, _CH)], src_a)
    pltpu.sync_copy(dst_r.at[pl.ds(e0, _CH)], dst_a)
    pltpu.async_copy(table.at[src_a], rows_a, sem_a)
    lax.fori_loop(0, _NCHUNK, chunk, 0)
    pltpu.make_async_copy(rows_a, acc.at[dst_a], ssem_a).wait()
    pltpu.make_async_copy(rows_b, acc.at[dst_b], ssem_b).wait()
    tb = e0 + _NCHUNK * _CH
    pltpu.sync_copy(src_r.at[pl.ds(tb, _TAIL)], src_t)
    pltpu.sync_copy(dst_r.at[pl.ds(tb, _TAIL)], dst_t)
    pltpu.async_copy(table.at[src_t], rows_t, sem_a).wait()
    pltpu.sync_copy(rows_t, acc.at[dst_t], add=True)
    plsc.subcore_barrier()
    writeout(out_sum)

    if with_deg:
      zero_acc()
      plsc.subcore_barrier()
      pltpu.sync_copy(ones_d, rows_b)

      def deg_step(j, d_cur, i_cur, sc_cur, d_nxt, i_nxt, sc_nxt):
        @pl.when(j < _NCHUNK - 1)
        def _():
          @pl.when(j >= 1)
          def _():
            pltpu.make_async_copy(rows_b, acc.at[d_nxt], sc_nxt).wait()

          pltpu.async_copy(dst_r.at[pl.ds(e0 + (j + 1) * _CH, _CH)],
                           d_nxt, i_nxt)

        pltpu.make_async_copy(dst_r.at[pl.ds(e0, _CH)], d_cur, i_cur).wait()
        pltpu.async_copy(rows_b, acc.at[d_cur], sc_cur, add=True)

      def deg_chunk(j, carry):
        @pl.when(j % 2 == 0)
        def _():
          deg_step(j, dst_a, sem_a, ssem_a, dst_b, sem_b, ssem_b)

        @pl.when(j % 2 == 1)
        def _():
          deg_step(j, dst_b, sem_b, ssem_b, dst_a, sem_a, ssem_a)

        return carry

      pltpu.async_copy(dst_r.at[pl.ds(e0, _CH)], dst_a, sem_a)
      lax.fori_loop(0, _NCHUNK, deg_chunk, 0)
      pltpu.make_async_copy(rows_b, acc.at[dst_a], ssem_a).wait()
      pltpu.make_async_copy(rows_b, acc.at[dst_b], ssem_b).wait()
      pltpu.sync_copy(ones_d.at[pl.ds(0, _TAIL)], rows_t)
      pltpu.sync_copy(dst_r.at[pl.ds(tb, _TAIL)], dst_t)
      pltpu.sync_copy(rows_t, acc.at[dst_t], add=True)
      plsc.subcore_barrier()
      writeout(out_deg)

  return pl.kernel(body, mesh=mesh, out_type=out_type, scratch_types=scratch)


_EDGE_ACCUM_DEG = _make_edge_accum(True)
_EDGE_ACCUM = _make_edge_accum(False)


def _mm_body(x_ref, w_ref, o_ref):
  o_ref[...] = lax.dot_general(
      x_ref[...], w_ref[...], (((1,), (1,)), ((), ())),
      preferred_element_type=jnp.float32)


def _mm(x, w):
  return pl.pallas_call(
      _mm_body,
      grid=(_GRID,),
      in_specs=[
          pl.BlockSpec((_ROWS, _D), lambda i: (i, 0)),
          pl.BlockSpec((_D, _D), lambda i: (0, 0)),
      ],
      out_specs=pl.BlockSpec((_ROWS, _D), lambda i: (i, 0)),
      out_shape=jax.ShapeDtypeStruct((_N, _D), jnp.float32),
  )(x, w)


def _mid_body(s_ref, deg_ref, x_ref, w1r_ref, b1_ref, w2l_ref, w2r_ref,
              b2_ref, y2_ref, r2_ref):
  ssum = s_ref[0] + s_ref[1]
  deg = jnp.maximum(deg_ref[0] + deg_ref[1], 1.0)
  inv = (1.0 / deg)[:, 0:1]
  xr = lax.dot_general(x_ref[...], w1r_ref[...], (((1,), (1,)), ((), ())),
                       preferred_element_type=jnp.float32)
  h = jnp.maximum(ssum * inv + b1_ref[...] + xr, 0.0)
  y2_ref[...] = lax.dot_general(h, w2l_ref[...], (((1,), (1,)), ((), ())),
                                preferred_element_type=jnp.float32)
  r2_ref[...] = lax.dot_general(h, w2r_ref[...], (((1,), (1,)), ((), ())),
                                preferred_element_type=jnp.float32) + b2_ref[...]


def _mid(sums1, degs, x, w1r, b1, w2l, w2r, b2):
  return pl.pallas_call(
      _mid_body,
      grid=(_GRID,),
      in_specs=[
          pl.BlockSpec((_NC, _ROWS, _D), lambda i: (0, i, 0)),
          pl.BlockSpec((_NC, _ROWS, _D), lambda i: (0, i, 0)),
          pl.BlockSpec((_ROWS, _D), lambda i: (i, 0)),
          pl.BlockSpec((_D, _D), lambda i: (0, 0)),
          pl.BlockSpec((1, _D), lambda i: (0, 0)),
          pl.BlockSpec((_D, _D), lambda i: (0, 0)),
          pl.BlockSpec((_D, _D), lambda i: (0, 0)),
          pl.BlockSpec((1, _D), lambda i: (0, 0)),
      ],
      out_specs=[
          pl.BlockSpec((_ROWS, _D), lambda i: (i, 0)),
          pl.BlockSpec((_ROWS, _D), lambda i: (i, 0)),
      ],
      out_shape=[
          jax.ShapeDtypeStruct((_N, _D), jnp.float32),
          jax.ShapeDtypeStruct((_N, _D), jnp.float32),
      ],
  )(sums1, degs, x, w1r, b1, w2l, w2r, b2)


def _out_body(s_ref, deg_ref, r2_ref, o_ref):
  deg = jnp.maximum(deg_ref[0] + deg_ref[1], 1.0)
  inv = (1.0 / deg)[:, 0:1]
  o_ref[...] = (s_ref[0] + s_ref[1]) * inv + r2_ref[...]


def _final(sums2, degs, r2):
  return pl.pallas_call(
      _out_body,
      grid=(_GRID,),
      in_specs=[
          pl.BlockSpec((_NC, _ROWS, _D), lambda i: (0, i, 0)),
          pl.BlockSpec((_NC, _ROWS, _D), lambda i: (0, i, 0)),
          pl.BlockSpec((_ROWS, _D), lambda i: (i, 0)),
      ],
      out_specs=pl.BlockSpec((_ROWS, _D), lambda i: (i, 0)),
      out_shape=jax.ShapeDtypeStruct((_N, _D), jnp.float32),
  )(sums2, degs, r2)


def kernel(x, edge_index, W1l, b1l, W1r, W2l, b2l, W2r):
  ei = edge_index.astype(jnp.int32)
  src_r = ei[0]
  dst_r = ei[1]
  zeros_d = jnp.zeros((_CH, _D), jnp.float32)
  ones_d = jnp.ones((_CH, _D), jnp.float32)
  b1 = b1l.reshape(1, _D)
  b2 = b2l.reshape(1, _D)

  y1 = _mm(x, W1l)
  sums1, degs = _EDGE_ACCUM_DEG(y1, src_r, dst_r, zeros_d, ones_d)
  y2, r2 = _mid(sums1, degs, x, W1r, b1, W2l, W2r, b2)
  sums2, = _EDGE_ACCUM(y2, src_r, dst_r, zeros_d, ones_d)
  return _final(sums2, degs, r2)


# speedup vs baseline: 1.3901x; 1.3901x over previous
import jax
import jax.numpy as jnp
from jax import lax
from jax.experimental import pallas as pl
from jax.experimental.pallas import tpu as pltpu
from jax.experimental.pallas import tpu_sc as plsc

_N = 10000
_E = 320000
_D = 128
_NC = 2
_NS = 16
_NW = _NC * _NS
_EPW = _E // _NW
_CH = 128
_NCHUNK = _EPW // _CH
_TAIL = _EPW - _NCHUNK * _CH
_RPT = 640
_NP = _RPT * _NS

_ROWS = 1000
_GRID = _N // _ROWS


def _make_edge_accum(with_deg):
  mesh = plsc.VectorSubcoreMesh(core_axis_name="c", subcore_axis_name="s")
  out_type = [jax.ShapeDtypeStruct((_NC, _NP, _D), jnp.float32)]
  scratch = [
      pltpu.VMEM((2, _CH), jnp.int32),
      pltpu.VMEM((2, _CH), jnp.int32),
      pltpu.VMEM((_CH, _D), jnp.float32),
      pltpu.VMEM((_CH, _D), jnp.float32),
      pltpu.VMEM_SHARED((_NP, _D), jnp.float32),
      pltpu.SemaphoreType.DMA,
      pltpu.SemaphoreType.DMA,
      pltpu.VMEM((2, _TAIL), jnp.int32),
      pltpu.VMEM((_TAIL, _D), jnp.float32),
  ]
  if with_deg:
    out_type.append(jax.ShapeDtypeStruct((_NC, _NP, _D), jnp.float32))

  def body(table, sd_main, sd_tail, zeros_d, ones_d, *rest):
    if with_deg:
      (out_sum, out_deg, sd_a, sd_b,
       rows_a, rows_b, acc, sem_a, sem_b, sd_t, rows_t) = rest
    else:
      (out_sum, sd_a, sd_b,
       rows_a, rows_b, acc, sem_a, sem_b, sd_t, rows_t) = rest
    c = lax.axis_index("c")
    s = lax.axis_index("s")
    wid = c * _NS + s
    r0 = s * _RPT
    c0 = wid * _NCHUNK

    def zero_acc():
      pltpu.sync_copy(zeros_d, rows_a)
      for k in range(_RPT // _CH):
        pltpu.sync_copy(rows_a, acc.at[pl.ds(r0 + k * _CH, _CH)])

    def writeout(dst_hbm):
      for k in range(_RPT // _CH):
        pltpu.sync_copy(acc.at[pl.ds(r0 + k * _CH, _CH)], rows_a)
        pltpu.sync_copy(rows_a, dst_hbm.at[c, pl.ds(r0 + k * _CH, _CH)])

    zero_acc()
    plsc.subcore_barrier()

    def step(j, sd_cur, r_cur, g_cur, sd_nxt, r_nxt, g_nxt):
      @pl.when(j < _NCHUNK - 1)
      def _():
        pltpu.sync_copy(sd_main.at[c0 + j + 1], sd_nxt)
        pltpu.async_copy(table.at[sd_nxt.at[0]], r_nxt, g_nxt)

      pltpu.make_async_copy(table.at[sd_cur.at[0]], r_cur, g_cur).wait()
      pltpu.sync_copy(r_cur, acc.at[sd_cur.at[1]], add=True)

    def chunk(j, carry):
      @pl.when(j % 2 == 0)
      def _():
        step(j, sd_a, rows_a, sem_a, sd_b, rows_b, sem_b)

      @pl.when(j % 2 == 1)
      def _():
        step(j, sd_b, rows_b, sem_b, sd_a, rows_a, sem_a)

      return carry

    pltpu.sync_copy(sd_main.at[c0], sd_a)
    pltpu.async_copy(table.at[sd_a.at[0]], rows_a, sem_a)
    lax.fori_loop(0, _NCHUNK, chunk, 0)
    pltpu.sync_copy(sd_tail.at[wid], sd_t)
    pltpu.async_copy(table.at[sd_t.at[0]], rows_t, sem_a).wait()
    pltpu.sync_copy(rows_t, acc.at[sd_t.at[1]], add=True)
    plsc.subcore_barrier()
    writeout(out_sum)

    if with_deg:
      zero_acc()
      plsc.subcore_barrier()
      pltpu.sync_copy(ones_d, rows_b)

      def deg_step(j, sd_cur, i_cur, sd_nxt, i_nxt):
        @pl.when(j < _NCHUNK - 1)
        def _():
          pltpu.async_copy(sd_main.at[c0 + j + 1], sd_nxt, i_nxt)

        pltpu.make_async_copy(sd_main.at[c0], sd_cur, i_cur).wait()
        pltpu.sync_copy(rows_b, acc.at[sd_cur.at[1]], add=True)

      def deg_chunk(j, carry):
        @pl.when(j % 2 == 0)
        def _():
          deg_step(j, sd_a, sem_a, sd_b, sem_b)

        @pl.when(j % 2 == 1)
        def _():
          deg_step(j, sd_b, sem_b, sd_a, sem_a)

        return carry

      pltpu.async_copy(sd_main.at[c0], sd_a, sem_a)
      lax.fori_loop(0, _NCHUNK, deg_chunk, 0)
      pltpu.sync_copy(ones_d.at[pl.ds(0, _TAIL)], rows_t)
      pltpu.sync_copy(rows_t, acc.at[sd_t.at[1]], add=True)
      plsc.subcore_barrier()
      writeout(out_deg)

  return pl.kernel(body, mesh=mesh, out_type=out_type, scratch_types=scratch)


_EDGE_ACCUM_DEG = _make_edge_accum(True)
_EDGE_ACCUM = _make_edge_accum(False)


def _mm_body(x_ref, w_ref, o_ref):
  o_ref[...] = lax.dot_general(
      x_ref[...], w_ref[...], (((1,), (1,)), ((), ())),
      preferred_element_type=jnp.float32)


def _mm(x, w):
  return pl.pallas_call(
      _mm_body,
      grid=(_GRID,),
      in_specs=[
          pl.BlockSpec((_ROWS, _D), lambda i: (i, 0)),
          pl.BlockSpec((_D, _D), lambda i: (0, 0)),
      ],
      out_specs=pl.BlockSpec((_ROWS, _D), lambda i: (i, 0)),
      out_shape=jax.ShapeDtypeStruct((_N, _D), jnp.float32),
  )(x, w)


def _mid_body(s_ref, deg_ref, x_ref, w1r_ref, b1_ref, w2l_ref, w2r_ref,
              b2_ref, y2_ref, r2_ref):
  ssum = s_ref[0] + s_ref[1]
  deg = jnp.maximum(deg_ref[0] + deg_ref[1], 1.0)
  inv = (1.0 / deg)[:, 0:1]
  xr = lax.dot_general(x_ref[...], w1r_ref[...], (((1,), (1,)), ((), ())),
                       preferred_element_type=jnp.float32)
  h = jnp.maximum(ssum * inv + b1_ref[...] + xr, 0.0)
  y2_ref[...] = lax.dot_general(h, w2l_ref[...], (((1,), (1,)), ((), ())),
                                preferred_element_type=jnp.float32)
  r2_ref[...] = lax.dot_general(h, w2r_ref[...], (((1,), (1,)), ((), ())),
                                preferred_element_type=jnp.float32) + b2_ref[...]


def _mid(sums1, degs, x, w1r, b1, w2l, w2r, b2):
  return pl.pallas_call(
      _mid_body,
      grid=(_GRID,),
      in_specs=[
          pl.BlockSpec((_NC, _ROWS, _D), lambda i: (0, i, 0)),
          pl.BlockSpec((_NC, _ROWS, _D), lambda i: (0, i, 0)),
          pl.BlockSpec((_ROWS, _D), lambda i: (i, 0)),
          pl.BlockSpec((_D, _D), lambda i: (0, 0)),
          pl.BlockSpec((1, _D), lambda i: (0, 0)),
          pl.BlockSpec((_D, _D), lambda i: (0, 0)),
          pl.BlockSpec((_D, _D), lambda i: (0, 0)),
          pl.BlockSpec((1, _D), lambda i: (0, 0)),
      ],
      out_specs=[
          pl.BlockSpec((_ROWS, _D), lambda i: (i, 0)),
          pl.BlockSpec((_ROWS, _D), lambda i: (i, 0)),
      ],
      out_shape=[
          jax.ShapeDtypeStruct((_N, _D), jnp.float32),
          jax.ShapeDtypeStruct((_N, _D), jnp.float32),
      ],
  )(sums1, degs, x, w1r, b1, w2l, w2r, b2)


def _out_body(s_ref, deg_ref, r2_ref, o_ref):
  deg = jnp.maximum(deg_ref[0] + deg_ref[1], 1.0)
  inv = (1.0 / deg)[:, 0:1]
  o_ref[...] = (s_ref[0] + s_ref[1]) * inv + r2_ref[...]


def _final(sums2, degs, r2):
  return pl.pallas_call(
      _out_body,
      grid=(_GRID,),
      in_specs=[
          pl.BlockSpec((_NC, _ROWS, _D), lambda i: (0, i, 0)),
          pl.BlockSpec((_NC, _ROWS, _D), lambda i: (0, i, 0)),
          pl.BlockSpec((_ROWS, _D), lambda i: (i, 0)),
      ],
      out_specs=pl.BlockSpec((_ROWS, _D), lambda i: (i, 0)),
      out_shape=jax.ShapeDtypeStruct((_N, _D), jnp.float32),
  )(sums2, degs, r2)


def kernel(x, edge_index, W1l, b1l, W1r, W2l, b2l, W2r):
  ei = edge_index.astype(jnp.int32).reshape(2, _NW, _EPW)
  main = ei[:, :, :_NCHUNK * _CH].reshape(2, _NW, _NCHUNK, _CH)
  sd_main = jnp.stack([main[0], main[1]], axis=2).reshape(
      _NW * _NCHUNK, 2, _CH)
  tail = ei[:, :, _NCHUNK * _CH:]
  sd_tail = jnp.stack([tail[0], tail[1]], axis=1)
  zeros_d = jnp.zeros((_CH, _D), jnp.float32)
  ones_d = jnp.ones((_CH, _D), jnp.float32)
  b1 = b1l.reshape(1, _D)
  b2 = b2l.reshape(1, _D)

  y1 = _mm(x, W1l)
  sums1, degs = _EDGE_ACCUM_DEG(y1, sd_main, sd_tail, zeros_d, ones_d)
  y2, r2 = _mid(sums1, degs, x, W1r, b1, W2l, W2r, b2)
  sums2, = _EDGE_ACCUM(y2, sd_main, sd_tail, zeros_d, ones_d)
  return _final(sums2, degs, r2)


# speedup vs baseline: 1.4125x; 1.0161x over previous
import jax
import jax.numpy as jnp
from jax import lax
from jax.experimental import pallas as pl
from jax.experimental.pallas import tpu as pltpu
from jax.experimental.pallas import tpu_sc as plsc

_N = 10000
_E = 320000
_D = 128
_NC = 2
_NS = 16
_NW = _NC * _NS
_EPW = _E // _NW
_CH = 128
_NCHUNK = _EPW // _CH
_TAIL = _EPW - _NCHUNK * _CH
_RPT = 640
_NP = _RPT * _NS

_ROWS = 1000
_GRID = _N // _ROWS


def _make_edge_accum(with_deg):
  mesh = plsc.VectorSubcoreMesh(core_axis_name="c", subcore_axis_name="s")
  out_type = [jax.ShapeDtypeStruct((_NC, _NP, _D), jnp.float32)]
  scratch = [
      pltpu.VMEM((2, _CH), jnp.int32),
      pltpu.VMEM((2, _CH), jnp.int32),
      pltpu.VMEM((_CH, _D), jnp.float32),
      pltpu.VMEM((_CH, _D), jnp.float32),
      pltpu.VMEM_SHARED((_NP, _D), jnp.float32),
      pltpu.SemaphoreType.DMA,
      pltpu.SemaphoreType.DMA,
      pltpu.SemaphoreType.DMA,
      pltpu.SemaphoreType.DMA,
      pltpu.VMEM((2, _TAIL), jnp.int32),
      pltpu.VMEM((_TAIL, _D), jnp.float32),
  ]
  if with_deg:
    out_type.append(jax.ShapeDtypeStruct((_NC, _NP, _D), jnp.float32))

  def body(table, sd_main, sd_tail, zeros_d, ones_d, *rest):
    if with_deg:
      (out_sum, out_deg, sd_a, sd_b, rows_a, rows_b, acc,
       sem_a, sem_b, isem_a, isem_b, sd_t, rows_t) = rest
    else:
      (out_sum, sd_a, sd_b, rows_a, rows_b, acc,
       sem_a, sem_b, isem_a, isem_b, sd_t, rows_t) = rest
    c = lax.axis_index("c")
    s = lax.axis_index("s")
    wid = c * _NS + s
    r0 = s * _RPT
    c0 = wid * _NCHUNK

    def zero_acc():
      pltpu.sync_copy(zeros_d, rows_a)
      cps = [pltpu.async_copy(rows_a, acc.at[pl.ds(r0 + k * _CH, _CH)], sem_a)
             for k in range(_RPT // _CH)]
      for cp in cps:
        cp.wait()

    def writeout(dst_hbm):
      for k in range(_RPT // _CH):
        buf = rows_a if k % 2 == 0 else rows_b
        sm = sem_a if k % 2 == 0 else sem_b
        if k >= 2:
          pltpu.make_async_copy(
              buf, dst_hbm.at[c, pl.ds(r0, _CH)], sm).wait()
        pltpu.sync_copy(acc.at[pl.ds(r0 + k * _CH, _CH)], buf)
        pltpu.async_copy(buf, dst_hbm.at[c, pl.ds(r0 + k * _CH, _CH)], sm)
      pltpu.make_async_copy(rows_a, dst_hbm.at[c, pl.ds(r0, _CH)], sem_a).wait()
      pltpu.make_async_copy(rows_b, dst_hbm.at[c, pl.ds(r0, _CH)], sem_b).wait()

    zero_acc()
    plsc.subcore_barrier()

    def step(j, sd_cur, r_cur, g_cur, i_cur, sd_nxt, r_nxt, g_nxt, i_nxt):
      @pl.when(j < _NCHUNK - 1)
      def _():
        pltpu.make_async_copy(sd_main.at[c0], sd_nxt, i_nxt).wait()
        pltpu.async_copy(table.at[sd_nxt.at[0]], r_nxt, g_nxt)

      pltpu.make_async_copy(table.at[sd_cur.at[0]], r_cur, g_cur).wait()
      pltpu.sync_copy(r_cur, acc.at[sd_cur.at[1]], add=True)

      @pl.when(j < _NCHUNK - 2)
      def _():
        pltpu.async_copy(sd_main.at[c0 + j + 2], sd_cur, i_cur)

    def chunk(j, carry):
      @pl.when(j % 2 == 0)
      def _():
        step(j, sd_a, rows_a, sem_a, isem_a, sd_b, rows_b, sem_b, isem_b)

      @pl.when(j % 2 == 1)
      def _():
        step(j, sd_b, rows_b, sem_b, isem_b, sd_a, rows_a, sem_a, isem_a)

      return carry

    pltpu.sync_copy(sd_main.at[c0], sd_a)
    pltpu.async_copy(table.at[sd_a.at[0]], rows_a, sem_a)
    pltpu.async_copy(sd_main.at[c0 + 1], sd_b, isem_b)
    lax.fori_loop(0, _NCHUNK, chunk, 0)
    pltpu.sync_copy(sd_tail.at[wid], sd_t)
    pltpu.async_copy(table.at[sd_t.at[0]], rows_t, sem_a).wait()
    pltpu.sync_copy(rows_t, acc.at[sd_t.at[1]], add=True)
    plsc.subcore_barrier()
    writeout(out_sum)

    if with_deg:
      zero_acc()
      plsc.subcore_barrier()
      pltpu.sync_copy(ones_d, rows_b)

      def deg_step(j, sd_cur, i_cur, sd_nxt, i_nxt):
        @pl.when(j < _NCHUNK - 1)
        def _():
          pltpu.async_copy(sd_main.at[c0 + j + 1], sd_nxt, i_nxt)

        pltpu.make_async_copy(sd_main.at[c0], sd_cur, i_cur).wait()
        pltpu.sync_copy(rows_b, acc.at[sd_cur.at[1]], add=True)

      def deg_chunk(j, carry):
        @pl.when(j % 2 == 0)
        def _():
          deg_step(j, sd_a, sem_a, sd_b, sem_b)

        @pl.when(j % 2 == 1)
        def _():
          deg_step(j, sd_b, sem_b, sd_a, sem_a)

        return carry

      pltpu.async_copy(sd_main.at[c0], sd_a, sem_a)
      lax.fori_loop(0, _NCHUNK, deg_chunk, 0)
      pltpu.sync_copy(ones_d.at[pl.ds(0, _TAIL)], rows_t)
      pltpu.sync_copy(rows_t, acc.at[sd_t.at[1]], add=True)
      plsc.subcore_barrier()
      writeout(out_deg)

  return pl.kernel(body, mesh=mesh, out_type=out_type, scratch_types=scratch)


_EDGE_ACCUM_DEG = _make_edge_accum(True)
_EDGE_ACCUM = _make_edge_accum(False)


def _mm_body(x_ref, w_ref, o_ref):
  o_ref[...] = lax.dot_general(
      x_ref[...], w_ref[...], (((1,), (1,)), ((), ())),
      preferred_element_type=jnp.float32)


def _mm(x, w):
  return pl.pallas_call(
      _mm_body,
      grid=(_GRID,),
      in_specs=[
          pl.BlockSpec((_ROWS, _D), lambda i: (i, 0)),
          pl.BlockSpec((_D, _D), lambda i: (0, 0)),
      ],
      out_specs=pl.BlockSpec((_ROWS, _D), lambda i: (i, 0)),
      out_shape=jax.ShapeDtypeStruct((_N, _D), jnp.float32),
  )(x, w)


def _mid_body(s_ref, deg_ref, x_ref, w1r_ref, b1_ref, w2l_ref, w2r_ref,
              b2_ref, y2_ref, r2_ref):
  ssum = s_ref[0] + s_ref[1]
  deg = jnp.maximum(deg_ref[0] + deg_ref[1], 1.0)
  inv = (1.0 / deg)[:, 0:1]
  xr = lax.dot_general(x_ref[...], w1r_ref[...], (((1,), (1,)), ((), ())),
                       preferred_element_type=jnp.float32)
  h = jnp.maximum(ssum * inv + b1_ref[...] + xr, 0.0)
  y2_ref[...] = lax.dot_general(h, w2l_ref[...], (((1,), (1,)), ((), ())),
                                preferred_element_type=jnp.float32)
  r2_ref[...] = lax.dot_general(h, w2r_ref[...], (((1,), (1,)), ((), ())),
                                preferred_element_type=jnp.float32) + b2_ref[...]


def _mid(sums1, degs, x, w1r, b1, w2l, w2r, b2):
  return pl.pallas_call(
      _mid_body,
      grid=(_GRID,),
      in_specs=[
          pl.BlockSpec((_NC, _ROWS, _D), lambda i: (0, i, 0)),
          pl.BlockSpec((_NC, _ROWS, _D), lambda i: (0, i, 0)),
          pl.BlockSpec((_ROWS, _D), lambda i: (i, 0)),
          pl.BlockSpec((_D, _D), lambda i: (0, 0)),
          pl.BlockSpec((1, _D), lambda i: (0, 0)),
          pl.BlockSpec((_D, _D), lambda i: (0, 0)),
          pl.BlockSpec((_D, _D), lambda i: (0, 0)),
          pl.BlockSpec((1, _D), lambda i: (0, 0)),
      ],
      out_specs=[
          pl.BlockSpec((_ROWS, _D), lambda i: (i, 0)),
          pl.BlockSpec((_ROWS, _D), lambda i: (i, 0)),
      ],
      out_shape=[
          jax.ShapeDtypeStruct((_N, _D), jnp.float32),
          jax.ShapeDtypeStruct((_N, _D), jnp.float32),
      ],
  )(sums1, degs, x, w1r, b1, w2l, w2r, b2)


def _out_body(s_ref, deg_ref, r2_ref, o_ref):
  deg = jnp.maximum(deg_ref[0] + deg_ref[1], 1.0)
  inv = (1.0 / deg)[:, 0:1]
  o_ref[...] = (s_ref[0] + s_ref[1]) * inv + r2_ref[...]


def _final(sums2, degs, r2):
  return pl.pallas_call(
      _out_body,
      grid=(_GRID,),
      in_specs=[
          pl.BlockSpec((_NC, _ROWS, _D), lambda i: (0, i, 0)),
          pl.BlockSpec((_NC, _ROWS, _D), lambda i: (0, i, 0)),
          pl.BlockSpec((_ROWS, _D), lambda i: (i, 0)),
      ],
      out_specs=pl.BlockSpec((_ROWS, _D), lambda i: (i, 0)),
      out_shape=jax.ShapeDtypeStruct((_N, _D), jnp.float32),
  )(sums2, degs, r2)


def kernel(x, edge_index, W1l, b1l, W1r, W2l, b2l, W2r):
  ei = edge_index.astype(jnp.int32).reshape(2, _NW, _EPW)
  main = ei[:, :, :_NCHUNK * _CH].reshape(2, _NW, _NCHUNK, _CH)
  sd_main = jnp.stack([main[0], main[1]], axis=2).reshape(
      _NW * _NCHUNK, 2, _CH)
  tail = ei[:, :, _NCHUNK * _CH:]
  sd_tail = jnp.stack([tail[0], tail[1]], axis=1)
  zeros_d = jnp.zeros((_CH, _D), jnp.float32)
  ones_d = jnp.ones((_CH, _D), jnp.float32)
  b1 = b1l.reshape(1, _D)
  b2 = b2l.reshape(1, _D)

  y1 = _mm(x, W1l)
  sums1, degs = _EDGE_ACCUM_DEG(y1, sd_main, sd_tail, zeros_d, ones_d)
  y2, r2 = _mid(sums1, degs, x, W1r, b1, W2l, W2r, b2)
  sums2, = _EDGE_ACCUM(y2, sd_main, sd_tail, zeros_d, ones_d)
  return _final(sums2, degs, r2)
